# baseline (speedup 1.0000x reference)
import functools

import jax
import jax.numpy as jnp
from jax import lax
from jax.experimental import pallas as pl
from jax.experimental.pallas import tpu as pltpu
from jax.experimental.pallas import tpu_sc as plsc

K_NEIGHBORS = 9
EPS = 1e-5
_HI = jax.lax.Precision.HIGHEST
_NTOT = float(16 * 1024)

_NC, _NS, _L = 2, 16, 16
_NW = _NC * _NS


def _norm(h, stats, g, be):
    m = stats[0:1, :] / _NTOT
    v = stats[1:2, :] / _NTOT - m * m
    return (h - m) / jnp.sqrt(v + EPS) * g + be


def _fc1_body(x_ref, w_ref, b_ref, h_ref, s_ref):
    h = jnp.dot(x_ref[...], w_ref[...],
                preferred_element_type=jnp.float32) + b_ref[...]
    h_ref[...] = h

    @pl.when(pl.program_id(0) == 0)
    def _():
        s_ref[...] = jnp.zeros_like(s_ref)

    s_ref[0:1, :] += jnp.sum(h, axis=0, keepdims=True)
    s_ref[1:2, :] += jnp.sum(h * h, axis=0, keepdims=True)


def _knn_idx_body(h_ref, s_ref, g_ref, be_ref, hn_ref, idx_ref):
    hb = _norm(h_ref[0], s_ref[...], g_ref[...], be_ref[...])
    hn_ref[0] = jnp.concatenate(
        [hb, jnp.zeros((hb.shape[0], 128 - hb.shape[1]), jnp.float32)], axis=1)
    n = hb.shape[0]
    sq = jnp.sum(hb * hb, axis=1, keepdims=True)
    gram = jax.lax.dot_general(hb, hb, (((1,), (1,)), ((), ())))
    dist = sq - 2.0 * gram + sq.T
    cols = jax.lax.broadcasted_iota(jnp.int32, (n, n), 1)
    d = dist
    ams = []
    for _ in range(K_NEIGHBORS):
        mv = jnp.min(d, axis=1, keepdims=True)
        am = jnp.min(jnp.where(d == mv, cols, n), axis=1, keepdims=True)
        ams.append(am)
        d = jnp.where(cols == am, 3e38, d)
    idx_ref[0] = jnp.concatenate(ams, axis=1) + pl.program_id(0) * n


def _make_sc_agg(R, C):
    npw = R // _NW
    ch_nodes = 8
    nch = npw // ch_nodes
    ch_idx = ch_nodes * K_NEIGHBORS
    cl = C // _L
    mesh = plsc.VectorSubcoreMesh(core_axis_name="c", subcore_axis_name="s")

    nbuf = 4
    assert nch % nbuf == 0

    @functools.partial(
        pl.kernel,
        out_type=jax.ShapeDtypeStruct((R, 128), jnp.float32),
        mesh=mesh,
        scratch_types=[
            pltpu.VMEM((nch, ch_idx), jnp.int32),
            pltpu.VMEM((nbuf, ch_idx, 128), jnp.float32),
            pltpu.VMEM((nbuf, ch_nodes, 128), jnp.float32),
            pltpu.VMEM((nbuf, ch_nodes, 128), jnp.float32),
            pltpu.SemaphoreType.DMA((nbuf,)),
            pltpu.SemaphoreType.DMA((nbuf,)),
            pltpu.SemaphoreType.DMA((nbuf,)),
        ],
    )
    def sc_agg(h_hbm, idx_hbm, out_hbm, idx_v, rows_v, own_v, o_v,
               sem_g, sem_o, sem_s):
        wid = lax.axis_index("s") * _NC + lax.axis_index("c")
        base = wid * npw
        pltpu.sync_copy(idx_hbm.at[wid], idx_v)

        def issue(ch, buf):
            pltpu.async_copy(h_hbm.at[idx_v.at[ch]], rows_v.at[buf],
                             sem_g.at[buf])
            pltpu.async_copy(h_hbm.at[pl.ds(base + ch * ch_nodes, ch_nodes)],
                             own_v.at[buf], sem_o.at[buf])

        def wait_in(ch, buf):
            pltpu.make_async_copy(h_hbm.at[idx_v.at[ch]], rows_v.at[buf],
                                  sem_g.at[buf]).wait()
            pltpu.make_async_copy(
                h_hbm.at[pl.ds(base + ch * ch_nodes, ch_nodes)],
                own_v.at[buf], sem_o.at[buf]).wait()

        def store(ch, buf):
            pltpu.async_copy(o_v.at[buf],
                             out_hbm.at[pl.ds(base + ch * ch_nodes, ch_nodes)],
                             sem_s.at[buf])

        def wait_store(ch, buf):
            pltpu.make_async_copy(
                o_v.at[buf],
                out_hbm.at[pl.ds(base + ch * ch_nodes, ch_nodes)],
                sem_s.at[buf]).wait()

        def compute(buf):
            rv = rows_v.at[buf]
            ov = own_v.at[buf]
            for i in range(ch_nodes):
                for c in range(cl):
                    s = pl.ds(c * _L, _L)
                    acc = rv[i * K_NEIGHBORS, s]
                    for k in range(1, K_NEIGHBORS):
                        acc = jnp.maximum(acc, rv[i * K_NEIGHBORS + k, s])
                    o_v[buf, i, s] = acc - ov[i, s]

        for b0 in range(nbuf - 1):
            issue(b0, b0)

        @pl.loop(0, nch, step=nbuf)
        def _(ch0):
            for b in range(nbuf):
                cur = ch0 + b

                @pl.when(cur + nbuf - 1 < nch)
                def _():
                    issue(cur + nbuf - 1, (b + nbuf - 1) % nbuf)

                @pl.when(cur >= nbuf)
                def _():
                    wait_store(cur - nbuf, b)

                wait_in(cur, b)
                compute(b)
                store(cur, b)

        for b in range(nbuf):
            last = nch - nbuf + b
            wait_store(last, b)

    return sc_agg


def _fcg_body(h_ref, a_ref, wg_ref, bg_ref, o_ref, s_ref):
    c = wg_ref.shape[1]
    o = (jnp.dot(h_ref[:, :c], wg_ref[:c, :], preferred_element_type=jnp.float32,
                 precision=_HI)
         + jnp.dot(a_ref[:, :c], wg_ref[c:, :], preferred_element_type=jnp.float32,
                   precision=_HI)
         + bg_ref[...])
    o_ref[...] = o

    @pl.when(pl.program_id(0) == 0)
    def _():
        s_ref[...] = jnp.zeros_like(s_ref)

    s_ref[0:1, :] += jnp.sum(o, axis=0, keepdims=True)
    s_ref[1:2, :] += jnp.sum(o * o, axis=0, keepdims=True)


def _fc2_body(o_ref, sg_ref, gg_ref, beg_ref, w2_ref, b2_ref, o2_ref, s_ref):
    o = _norm(o_ref[...], sg_ref[...], gg_ref[...], beg_ref[...])
    o = 0.5 * o * (1.0 + jax.lax.erf(o * 0.7071067811865476))
    o2 = jnp.dot(o, w2_ref[...], preferred_element_type=jnp.float32,
                 precision=_HI) + b2_ref[...]
    o2_ref[...] = o2

    @pl.when(pl.program_id(0) == 0)
    def _():
        s_ref[...] = jnp.zeros_like(s_ref)

    s_ref[0:1, :] += jnp.sum(o2, axis=0, keepdims=True)
    s_ref[1:2, :] += jnp.sum(o2 * o2, axis=0, keepdims=True)


def _bn_res_body(o2_ref, s2_ref, g2_ref, be2_ref, x_ref, out_ref):
    out_ref[...] = (_norm(o2_ref[...], s2_ref[...], g2_ref[...], be2_ref[...])
                    + x_ref[...])


def kernel(x, W1, b1, g1, be1, Wg, bg, gg, beg, W2, b2, g2, be2):
    B, N, C = x.shape
    R = B * N
    NB = 16
    RB = R // NB
    xf = x.reshape(R, C)
    row = lambda v: v.reshape(1, -1)

    rows_spec = pl.BlockSpec((RB, C), lambda i: (i, 0))
    stat_spec = pl.BlockSpec((2, C), lambda i: (0, 0))
    vec_spec = pl.BlockSpec((1, C), lambda i: (0, 0))
    full = lambda a: pl.BlockSpec(a.shape, lambda i: tuple(0 for _ in a.shape))

    hraw, s1 = pl.pallas_call(
        _fc1_body,
        grid=(NB,),
        in_specs=[rows_spec, full(W1), vec_spec],
        out_specs=[rows_spec, stat_spec],
        out_shape=[jax.ShapeDtypeStruct((R, C), jnp.float32),
                   jax.ShapeDtypeStruct((2, C), jnp.float32)],
    )(xf, W1, row(b1))

    hn, idx = pl.pallas_call(
        _knn_idx_body,
        grid=(B,),
        in_specs=[pl.BlockSpec((1, N, C), lambda b: (b, 0, 0)),
                  stat_spec, vec_spec, vec_spec],
        out_specs=[pl.BlockSpec((1, N, 128), lambda b: (b, 0, 0)),
                   pl.BlockSpec((1, N, K_NEIGHBORS), lambda b: (b, 0, 0))],
        out_shape=[jax.ShapeDtypeStruct((B, N, 128), jnp.float32),
                   jax.ShapeDtypeStruct((B, N, K_NEIGHBORS), jnp.int32)],
    )(hraw.reshape(B, N, C), s1, row(g1), row(be1))

    hn2 = hn.reshape(R, 128)
    idx_w = idx.reshape(_NW, (R // _NW) // 8, 8 * K_NEIGHBORS)
    agg = _make_sc_agg(R, C)(hn2, idx_w)

    wide_spec = pl.BlockSpec((RB, 128), lambda i: (i, 0))
    o1, sg = pl.pallas_call(
        _fcg_body,
        grid=(NB,),
        in_specs=[wide_spec, wide_spec, full(Wg), vec_spec],
        out_specs=[rows_spec, stat_spec],
        out_shape=[jax.ShapeDtypeStruct((R, C), jnp.float32),
                   jax.ShapeDtypeStruct((2, C), jnp.float32)],
    )(hn2, agg, Wg, row(bg))

    o2, s2 = pl.pallas_call(
        _fc2_body,
        grid=(NB,),
        in_specs=[rows_spec, stat_spec, vec_spec, vec_spec, full(W2), vec_spec],
        out_specs=[rows_spec, stat_spec],
        out_shape=[jax.ShapeDtypeStruct((R, C), jnp.float32),
                   jax.ShapeDtypeStruct((2, C), jnp.float32)],
    )(o1, sg, row(gg), row(beg), W2, row(b2))

    out = pl.pallas_call(
        _bn_res_body,
        grid=(NB,),
        in_specs=[rows_spec, stat_spec, vec_spec, vec_spec, rows_spec],
        out_specs=rows_spec,
        out_shape=jax.ShapeDtypeStruct((R, C), jnp.float32),
    )(o2, s2, row(g2), row(be2), xf)
    return out.reshape(B, N, C)


# speedup vs baseline: 1.1003x; 1.1003x over previous
import functools

import jax
import jax.numpy as jnp
from jax import lax
from jax.experimental import pallas as pl
from jax.experimental.pallas import tpu as pltpu
from jax.experimental.pallas import tpu_sc as plsc

K_NEIGHBORS = 9
EPS = 1e-5
_HI = jax.lax.Precision.HIGHEST
_NTOT = float(16 * 1024)

_NC, _NS, _L = 2, 16, 16
_NW = _NC * _NS


def _norm(h, stats, g, be):
    m = stats[0:1, :] / _NTOT
    v = stats[1:2, :] / _NTOT - m * m
    return (h - m) / jnp.sqrt(v + EPS) * g + be


def _fc1_body(x_ref, w_ref, b_ref, h_ref, s_ref):
    h = jnp.dot(x_ref[...], w_ref[...],
                preferred_element_type=jnp.float32) + b_ref[...]
    h_ref[...] = h

    @pl.when(pl.program_id(0) == 0)
    def _():
        s_ref[...] = jnp.zeros_like(s_ref)

    s_ref[0:1, :] += jnp.sum(h, axis=0, keepdims=True)
    s_ref[1:2, :] += jnp.sum(h * h, axis=0, keepdims=True)


def _knn_idx_body(h_ref, s_ref, g_ref, be_ref, hn_ref, idx_ref):
    hb = _norm(h_ref[0], s_ref[...], g_ref[...], be_ref[...])
    hn_ref[0] = jnp.concatenate(
        [hb, jnp.zeros((hb.shape[0], 128 - hb.shape[1]), jnp.float32)], axis=1)
    n = hb.shape[0]
    sq = jnp.sum(hb * hb, axis=1, keepdims=True)
    gram = jax.lax.dot_general(hb, hb, (((1,), (1,)), ((), ())))
    dist = sq - 2.0 * gram + sq.T
    cols = jax.lax.broadcasted_iota(jnp.int32, (n, n), 1)
    d = dist
    ams = []
    for _ in range(K_NEIGHBORS):
        mv = jnp.min(d, axis=1, keepdims=True)
        am = jnp.min(jnp.where(d == mv, cols, n), axis=1, keepdims=True)
        ams.append(am)
        d = jnp.where(cols == am, 3e38, d)
    idx_ref[0] = jnp.concatenate(ams, axis=1) + pl.program_id(0) * n


def _make_sc_agg(R, C):
    npw = R // _NW
    ch_nodes = 8
    nch = npw // ch_nodes
    ch_idx = ch_nodes * K_NEIGHBORS
    cl = C // _L
    mesh = plsc.VectorSubcoreMesh(core_axis_name="c", subcore_axis_name="s")

    nbuf = 4
    assert nch % nbuf == 0

    @functools.partial(
        pl.kernel,
        out_type=jax.ShapeDtypeStruct((R, 128), jnp.float32),
        mesh=mesh,
        scratch_types=[
            pltpu.VMEM((nch, ch_idx), jnp.int32),
            pltpu.VMEM((nbuf, ch_idx, 128), jnp.float32),
            pltpu.VMEM((nbuf, ch_nodes, 128), jnp.float32),
            pltpu.VMEM((nbuf, ch_nodes, 128), jnp.float32),
            pltpu.SemaphoreType.DMA((nbuf,)),
            pltpu.SemaphoreType.DMA((nbuf,)),
            pltpu.SemaphoreType.DMA((nbuf,)),
        ],
    )
    def sc_agg(h_hbm, idx_hbm, out_hbm, idx_v, rows_v, own_v, o_v,
               sem_g, sem_o, sem_s):
        wid = lax.axis_index("s") * _NC + lax.axis_index("c")
        base = wid * npw
        pltpu.sync_copy(idx_hbm.at[wid], idx_v)

        def issue(ch, buf):
            pltpu.async_copy(h_hbm.at[idx_v.at[ch]], rows_v.at[buf],
                             sem_g.at[buf])
            pltpu.async_copy(h_hbm.at[pl.ds(base + ch * ch_nodes, ch_nodes)],
                             own_v.at[buf], sem_o.at[buf])

        def wait_in(ch, buf):
            pltpu.make_async_copy(h_hbm.at[idx_v.at[ch]], rows_v.at[buf],
                                  sem_g.at[buf]).wait()
            pltpu.make_async_copy(
                h_hbm.at[pl.ds(base + ch * ch_nodes, ch_nodes)],
                own_v.at[buf], sem_o.at[buf]).wait()

        def store(ch, buf):
            pltpu.async_copy(o_v.at[buf],
                             out_hbm.at[pl.ds(base + ch * ch_nodes, ch_nodes)],
                             sem_s.at[buf])

        def wait_store(ch, buf):
            pltpu.make_async_copy(
                o_v.at[buf],
                out_hbm.at[pl.ds(base + ch * ch_nodes, ch_nodes)],
                sem_s.at[buf]).wait()

        def compute(buf):
            rv = rows_v.at[buf]
            ov = own_v.at[buf]
            for i in range(ch_nodes):
                for c in range(cl):
                    s = pl.ds(c * _L, _L)
                    acc = rv[i * K_NEIGHBORS, s]
                    for k in range(1, K_NEIGHBORS):
                        acc = jnp.maximum(acc, rv[i * K_NEIGHBORS + k, s])
                    o_v[buf, i, s] = acc - ov[i, s]

        for b0 in range(nbuf - 1):
            issue(b0, b0)

        @pl.loop(0, nch, step=nbuf)
        def _(ch0):
            for b in range(nbuf):
                cur = ch0 + b

                @pl.when(cur + nbuf - 1 < nch)
                def _():
                    issue(cur + nbuf - 1, (b + nbuf - 1) % nbuf)

                @pl.when(cur >= nbuf)
                def _():
                    wait_store(cur - nbuf, b)

                wait_in(cur, b)
                compute(b)
                store(cur, b)

        for b in range(nbuf):
            last = nch - nbuf + b
            wait_store(last, b)

    return sc_agg


def _fcg_body(h_ref, a_ref, wg_ref, bg_ref, o_ref, s_ref):
    c = wg_ref.shape[1]
    o = (jnp.dot(h_ref[:, :c], wg_ref[:c, :], preferred_element_type=jnp.float32,
                 precision=_HI)
         + jnp.dot(a_ref[:, :c], wg_ref[c:, :], preferred_element_type=jnp.float32,
                   precision=_HI)
         + bg_ref[...])
    o_ref[...] = o

    @pl.when(pl.program_id(0) == 0)
    def _():
        s_ref[...] = jnp.zeros_like(s_ref)

    s_ref[0:1, :] += jnp.sum(o, axis=0, keepdims=True)
    s_ref[1:2, :] += jnp.sum(o * o, axis=0, keepdims=True)


def _fc2_body(o_ref, sg_ref, gg_ref, beg_ref, w2_ref, b2_ref, o2_ref, s_ref):
    o = _norm(o_ref[...], sg_ref[...], gg_ref[...], beg_ref[...])
    o = 0.5 * o * (1.0 + jax.lax.erf(o * 0.7071067811865476))
    o2 = jnp.dot(o, w2_ref[...], preferred_element_type=jnp.float32,
                 precision=_HI) + b2_ref[...]
    o2_ref[...] = o2

    @pl.when(pl.program_id(0) == 0)
    def _():
        s_ref[...] = jnp.zeros_like(s_ref)

    s_ref[0:1, :] += jnp.sum(o2, axis=0, keepdims=True)
    s_ref[1:2, :] += jnp.sum(o2 * o2, axis=0, keepdims=True)


def _bn_res_body(o2_ref, s2_ref, g2_ref, be2_ref, x_ref, out_ref):
    out_ref[...] = (_norm(o2_ref[...], s2_ref[...], g2_ref[...], be2_ref[...])
                    + x_ref[...])


def kernel(x, W1, b1, g1, be1, Wg, bg, gg, beg, W2, b2, g2, be2):
    B, N, C = x.shape
    R = B * N
    NB = 16
    RB = R // NB
    xf = x.reshape(R, C)
    row = lambda v: v.reshape(1, -1)

    rows_spec = pl.BlockSpec((RB, C), lambda i: (i, 0))
    stat_spec = pl.BlockSpec((2, C), lambda i: (0, 0))
    vec_spec = pl.BlockSpec((1, C), lambda i: (0, 0))
    full = lambda a: pl.BlockSpec(a.shape, lambda i: tuple(0 for _ in a.shape))

    hraw, s1 = pl.pallas_call(
        _fc1_body,
        grid=(NB,),
        in_specs=[rows_spec, full(W1), vec_spec],
        out_specs=[rows_spec, stat_spec],
        out_shape=[jax.ShapeDtypeStruct((R, C), jnp.float32),
                   jax.ShapeDtypeStruct((2, C), jnp.float32)],
    )(xf, W1, row(b1))

    BH = B // 2
    RH = R // 2
    hr3 = hraw.reshape(B, N, C)
    halves = []
    for off in (0, BH):
        hn_h, idx_h = pl.pallas_call(
            _knn_idx_body,
            grid=(BH,),
            in_specs=[pl.BlockSpec((1, N, C), lambda b, o=off: (b + o, 0, 0)),
                      stat_spec, vec_spec, vec_spec],
            out_specs=[pl.BlockSpec((1, N, 128), lambda b: (b, 0, 0)),
                       pl.BlockSpec((1, N, K_NEIGHBORS), lambda b: (b, 0, 0))],
            out_shape=[jax.ShapeDtypeStruct((BH, N, 128), jnp.float32),
                       jax.ShapeDtypeStruct((BH, N, K_NEIGHBORS), jnp.int32)],
        )(hr3, s1, row(g1), row(be1))
        hn_h = hn_h.reshape(RH, 128)
        idx_w = idx_h.reshape(_NW, (RH // _NW) // 8, 8 * K_NEIGHBORS)
        halves.append((hn_h, _make_sc_agg(RH, C)(hn_h, idx_w)))

    hn2 = jnp.concatenate([halves[0][0], halves[1][0]], axis=0)
    agg = jnp.concatenate([halves[0][1], halves[1][1]], axis=0)

    wide_spec = pl.BlockSpec((RB, 128), lambda i: (i, 0))
    o1, sg = pl.pallas_call(
        _fcg_body,
        grid=(NB,),
        in_specs=[wide_spec, wide_spec, full(Wg), vec_spec],
        out_specs=[rows_spec, stat_spec],
        out_shape=[jax.ShapeDtypeStruct((R, C), jnp.float32),
                   jax.ShapeDtypeStruct((2, C), jnp.float32)],
    )(hn2, agg, Wg, row(bg))

    o2, s2 = pl.pallas_call(
        _fc2_body,
        grid=(NB,),
        in_specs=[rows_spec, stat_spec, vec_spec, vec_spec, full(W2), vec_spec],
        out_specs=[rows_spec, stat_spec],
        out_shape=[jax.ShapeDtypeStruct((R, C), jnp.float32),
                   jax.ShapeDtypeStruct((2, C), jnp.float32)],
    )(o1, sg, row(gg), row(beg), W2, row(b2))

    out = pl.pallas_call(
        _bn_res_body,
        grid=(NB,),
        in_specs=[rows_spec, stat_spec, vec_spec, vec_spec, rows_spec],
        out_specs=rows_spec,
        out_shape=jax.ShapeDtypeStruct((R, C), jnp.float32),
    )(o2, s2, row(g2), row(be2), xf)
    return out.reshape(B, N, C)


# speedup vs baseline: 1.3355x; 1.2137x over previous
import functools

import jax
import jax.numpy as jnp
from jax import lax
from jax.experimental import pallas as pl
from jax.experimental.pallas import tpu as pltpu
from jax.experimental.pallas import tpu_sc as plsc

K_NEIGHBORS = 9
EPS = 1e-5
_HI = jax.lax.Precision.HIGHEST
_NTOT = float(16 * 1024)

_NC, _NS, _L = 2, 16, 16
_NW = _NC * _NS


def _norm(h, stats, g, be):
    m = stats[0:1, :] / _NTOT
    v = stats[1:2, :] / _NTOT - m * m
    return (h - m) / jnp.sqrt(v + EPS) * g + be


def _fc1_body(x_ref, w_ref, b_ref, h_ref, s_ref):
    h = jnp.dot(x_ref[...], w_ref[...],
                preferred_element_type=jnp.float32) + b_ref[...]
    h_ref[...] = h

    @pl.when(pl.program_id(0) == 0)
    def _():
        s_ref[...] = jnp.zeros_like(s_ref)

    s_ref[0:1, :] += jnp.sum(h, axis=0, keepdims=True)
    s_ref[1:2, :] += jnp.sum(h * h, axis=0, keepdims=True)


def _knn_idx_body(h_ref, s_ref, g_ref, be_ref, hn_ref, idx_ref):
    hb = _norm(h_ref[0], s_ref[...], g_ref[...], be_ref[...])
    hn_ref[0] = jnp.concatenate(
        [hb, jnp.zeros((hb.shape[0], 128 - hb.shape[1]), jnp.float32)], axis=1)
    n = hb.shape[0]
    sq = jnp.sum(hb * hb, axis=1, keepdims=True)
    gram = jax.lax.dot_general(hb, hb, (((1,), (1,)), ((), ())))
    dist = sq - 2.0 * gram + sq.T
    cols = jax.lax.broadcasted_iota(jnp.int32, (n, n), 1).astype(jnp.float32)
    rows = jax.lax.broadcasted_iota(jnp.int32, (n, n), 0).astype(jnp.float32)
    d = jnp.where(cols == rows, 3e38, dist)
    ams = []
    for _ in range(K_NEIGHBORS - 1):
        mv = jnp.min(d, axis=1, keepdims=True)
        am = jnp.min(jnp.where(d == mv, cols, 2.0 * n), axis=1, keepdims=True)
        ams.append(am)
        d = jnp.where(cols == am, 3e38, d)
    idx_ref[0] = (jnp.concatenate(ams, axis=1).astype(jnp.int32)
                  + pl.program_id(0) * n)


def _make_sc_agg(R, C):
    kg = K_NEIGHBORS - 1
    npw = R // _NW
    ch_nodes = 8
    nch = npw // ch_nodes
    ch_idx = ch_nodes * kg
    cl = C // _L
    mesh = plsc.VectorSubcoreMesh(core_axis_name="c", subcore_axis_name="s")

    nbuf = 4
    assert nch % nbuf == 0

    @functools.partial(
        pl.kernel,
        out_type=jax.ShapeDtypeStruct((R, 128), jnp.float32),
        mesh=mesh,
        scratch_types=[
            pltpu.VMEM((nch, ch_idx), jnp.int32),
            pltpu.VMEM((nbuf, ch_idx, 128), jnp.float32),
            pltpu.VMEM((nbuf, ch_nodes, 128), jnp.float32),
            pltpu.VMEM((nbuf, ch_nodes, 128), jnp.float32),
            pltpu.SemaphoreType.DMA((nbuf,)),
            pltpu.SemaphoreType.DMA((nbuf,)),
            pltpu.SemaphoreType.DMA((nbuf,)),
        ],
    )
    def sc_agg(h_hbm, idx_hbm, out_hbm, idx_v, rows_v, own_v, o_v,
               sem_g, sem_o, sem_s):
        wid = lax.axis_index("s") * _NC + lax.axis_index("c")
        base = wid * npw
        pltpu.sync_copy(idx_hbm.at[wid], idx_v)

        def issue(ch, buf):
            pltpu.async_copy(h_hbm.at[idx_v.at[ch]], rows_v.at[buf],
                             sem_g.at[buf])
            pltpu.async_copy(h_hbm.at[pl.ds(base + ch * ch_nodes, ch_nodes)],
                             own_v.at[buf], sem_o.at[buf])

        def wait_in(ch, buf):
            pltpu.make_async_copy(h_hbm.at[idx_v.at[ch]], rows_v.at[buf],
                                  sem_g.at[buf]).wait()
            pltpu.make_async_copy(
                h_hbm.at[pl.ds(base + ch * ch_nodes, ch_nodes)],
                own_v.at[buf], sem_o.at[buf]).wait()

        def store(ch, buf):
            pltpu.async_copy(o_v.at[buf],
                             out_hbm.at[pl.ds(base + ch * ch_nodes, ch_nodes)],
                             sem_s.at[buf])

        def wait_store(ch, buf):
            pltpu.make_async_copy(
                o_v.at[buf],
                out_hbm.at[pl.ds(base + ch * ch_nodes, ch_nodes)],
                sem_s.at[buf]).wait()

        def compute(buf):
            rv = rows_v.at[buf]
            ov = own_v.at[buf]
            zero = jnp.zeros((_L,), jnp.float32)
            for i in range(ch_nodes):
                for c in range(cl):
                    s = pl.ds(c * _L, _L)
                    acc = rv[i * kg, s]
                    for k in range(1, kg):
                        acc = jnp.maximum(acc, rv[i * kg + k, s])
                    o_v[buf, i, s] = jnp.maximum(acc - ov[i, s], zero)

        for b0 in range(nbuf - 1):
            issue(b0, b0)

        @pl.loop(0, nch, step=nbuf)
        def _(ch0):
            for b in range(nbuf):
                cur = ch0 + b

                @pl.when(cur + nbuf - 1 < nch)
                def _():
                    issue(cur + nbuf - 1, (b + nbuf - 1) % nbuf)

                @pl.when(cur >= nbuf)
                def _():
                    wait_store(cur - nbuf, b)

                wait_in(cur, b)
                compute(b)
                store(cur, b)

        for b in range(nbuf):
            last = nch - nbuf + b
            wait_store(last, b)

    return sc_agg


def _fcg_body(ha_ref, hb_ref, aa_ref, ab_ref, wg_ref, bg_ref, o_ref, s_ref):
    c = wg_ref.shape[1]
    lo = pl.program_id(0) < (pl.num_programs(0) // 2)
    h = jnp.where(lo, ha_ref[:, :c], hb_ref[:, :c])
    a = jnp.where(lo, aa_ref[:, :c], ab_ref[:, :c])
    o = (jnp.dot(h, wg_ref[:c, :], preferred_element_type=jnp.float32,
                 precision=_HI)
         + jnp.dot(a, wg_ref[c:, :], preferred_element_type=jnp.float32,
                   precision=_HI)
         + bg_ref[...])
    o_ref[...] = o

    @pl.when(pl.program_id(0) == 0)
    def _():
        s_ref[...] = jnp.zeros_like(s_ref)

    s_ref[0:1, :] += jnp.sum(o, axis=0, keepdims=True)
    s_ref[1:2, :] += jnp.sum(o * o, axis=0, keepdims=True)


def _fc2_body(o_ref, sg_ref, gg_ref, beg_ref, w2_ref, b2_ref, o2_ref, s_ref):
    o = _norm(o_ref[...], sg_ref[...], gg_ref[...], beg_ref[...])
    o = 0.5 * o * (1.0 + jax.lax.erf(o * 0.7071067811865476))
    o2 = jnp.dot(o, w2_ref[...], preferred_element_type=jnp.float32,
                 precision=_HI) + b2_ref[...]
    o2_ref[...] = o2

    @pl.when(pl.program_id(0) == 0)
    def _():
        s_ref[...] = jnp.zeros_like(s_ref)

    s_ref[0:1, :] += jnp.sum(o2, axis=0, keepdims=True)
    s_ref[1:2, :] += jnp.sum(o2 * o2, axis=0, keepdims=True)


def _bn_res_body(o2_ref, s2_ref, g2_ref, be2_ref, x_ref, out_ref):
    out_ref[...] = (_norm(o2_ref[...], s2_ref[...], g2_ref[...], be2_ref[...])
                    + x_ref[...])


def kernel(x, W1, b1, g1, be1, Wg, bg, gg, beg, W2, b2, g2, be2):
    B, N, C = x.shape
    R = B * N
    NB = 16
    RB = R // NB
    xf = x.reshape(R, C)
    row = lambda v: v.reshape(1, -1)

    rows_spec = pl.BlockSpec((RB, C), lambda i: (i, 0))
    stat_spec = pl.BlockSpec((2, C), lambda i: (0, 0))
    vec_spec = pl.BlockSpec((1, C), lambda i: (0, 0))
    full = lambda a: pl.BlockSpec(a.shape, lambda i: tuple(0 for _ in a.shape))

    hraw, s1 = pl.pallas_call(
        _fc1_body,
        grid=(NB,),
        in_specs=[rows_spec, full(W1), vec_spec],
        out_specs=[rows_spec, stat_spec],
        out_shape=[jax.ShapeDtypeStruct((R, C), jnp.float32),
                   jax.ShapeDtypeStruct((2, C), jnp.float32)],
    )(xf, W1, row(b1))

    BH = B // 2
    RH = R // 2
    hr3 = hraw.reshape(B, N, C)
    halves = []
    for off in (0, BH):
        hn_h, idx_h = pl.pallas_call(
            _knn_idx_body,
            grid=(BH,),
            in_specs=[pl.BlockSpec((1, N, C), lambda b, o=off: (b + o, 0, 0)),
                      stat_spec, vec_spec, vec_spec],
            out_specs=[pl.BlockSpec((1, N, 128), lambda b: (b, 0, 0)),
                       pl.BlockSpec((1, N, K_NEIGHBORS - 1),
                                    lambda b: (b, 0, 0))],
            out_shape=[jax.ShapeDtypeStruct((BH, N, 128), jnp.float32),
                       jax.ShapeDtypeStruct((BH, N, K_NEIGHBORS - 1),
                                            jnp.int32)],
        )(hr3, s1, row(g1), row(be1))
        hn_h = hn_h.reshape(RH, 128)
        idx_w = idx_h.reshape(_NW, (RH // _NW) // 8, 8 * (K_NEIGHBORS - 1))
        halves.append((hn_h, _make_sc_agg(RH, C)(hn_h, idx_w)))

    nbh = NB // 2
    lo_spec = pl.BlockSpec((RB, 128), lambda i: (jnp.minimum(i, nbh - 1), 0))
    hi_spec = pl.BlockSpec((RB, 128), lambda i: (jnp.maximum(i - nbh, 0), 0))
    o1, sg = pl.pallas_call(
        _fcg_body,
        grid=(NB,),
        in_specs=[lo_spec, hi_spec, lo_spec, hi_spec, full(Wg), vec_spec],
        out_specs=[rows_spec, stat_spec],
        out_shape=[jax.ShapeDtypeStruct((R, C), jnp.float32),
                   jax.ShapeDtypeStruct((2, C), jnp.float32)],
    )(halves[0][0], halves[1][0], halves[0][1], halves[1][1], Wg, row(bg))

    o2, s2 = pl.pallas_call(
        _fc2_body,
        grid=(NB,),
        in_specs=[rows_spec, stat_spec, vec_spec, vec_spec, full(W2), vec_spec],
        out_specs=[rows_spec, stat_spec],
        out_shape=[jax.ShapeDtypeStruct((R, C), jnp.float32),
                   jax.ShapeDtypeStruct((2, C), jnp.float32)],
    )(o1, sg, row(gg), row(beg), W2, row(b2))

    out = pl.pallas_call(
        _bn_res_body,
        grid=(NB,),
        in_specs=[rows_spec, stat_spec, vec_spec, vec_spec, rows_spec],
        out_specs=rows_spec,
        out_shape=jax.ShapeDtypeStruct((R, C), jnp.float32),
    )(o2, s2, row(g2), row(be2), xf)
    return out.reshape(B, N, C)
